# baseline (speedup 1.0000x reference)
import functools

import jax
import jax.numpy as jnp
from jax.experimental import pallas as pl

_NUM_Q = 4
_K = 1024
_D = 64
_CCOST = 0.25
_BLK = 2048


def _rvq_kernel(z_ref, cb_ref, zrec_ref, idx_ref, loss_ref):
    step = pl.program_id(0)

    @pl.when(step == 0)
    def _init():
        loss_ref[...] = jnp.zeros((1, 1), jnp.float32)

    zin0 = z_ref[...]
    residual = zin0
    iota_k = jax.lax.broadcasted_iota(jnp.int32, (1, _K), 1)

    rec = jnp.zeros_like(zin0)
    loss_acc = jnp.float32(0.0)
    n_total = jnp.float32(zin0.shape[0] * _D)

    for i in range(_NUM_Q):
        W = cb_ref[i]
        zin = residual
        zsum = jnp.sum(zin * zin, axis=1, keepdims=True)
        wnorm = jnp.sum(W * W, axis=1)[None, :]
        mm = jax.lax.dot_general(
            zin, W, (((1,), (1,)), ((), ())),
            preferred_element_type=jnp.float32)
        d = (zsum + wnorm) - 2.0 * mm
        dmin = jnp.min(d, axis=1, keepdims=True)
        masked = jnp.where(d == dmin, iota_k, _K)
        idx = jnp.min(masked, axis=1)
        one_hot = (iota_k == idx[:, None]).astype(jnp.float32)
        zq = jax.lax.dot_general(
            one_hot, W, (((1,), (0,)), ((), ())),
            preferred_element_type=jnp.float32)
        zq_ste = zin + (zq - zin)
        diff = zin - zq
        loss_acc = loss_acc + (1.0 + _CCOST) * (
            jnp.sum(diff * diff) / n_total)
        residual = residual - zq_ste
        rec = rec + zq_ste if i else zq_ste
        idx_ref[:, i] = idx

    zrec_ref[...] = rec
    loss_ref[...] += jnp.full((1, 1), loss_acc, jnp.float32)


@jax.jit
def kernel(z, codebooks):
    n, d = z.shape
    num_blocks = n // _BLK
    grid = (num_blocks,)
    zrec, idx, loss = pl.pallas_call(
        _rvq_kernel,
        grid=grid,
        in_specs=[
            pl.BlockSpec((_BLK, d), lambda i: (i, 0)),
            pl.BlockSpec((_NUM_Q, _K, _D), lambda i: (0, 0, 0)),
        ],
        out_specs=[
            pl.BlockSpec((_BLK, d), lambda i: (i, 0)),
            pl.BlockSpec((_BLK, _NUM_Q), lambda i: (i, 0)),
            pl.BlockSpec((1, 1), lambda i: (0, 0)),
        ],
        out_shape=[
            jax.ShapeDtypeStruct((n, d), jnp.float32),
            jax.ShapeDtypeStruct((n, _NUM_Q), jnp.int32),
            jax.ShapeDtypeStruct((1, 1), jnp.float32),
        ],
    )(z, codebooks)
    total_loss = (loss[0, 0] * (_BLK / n)).astype(jnp.float32)
    return zrec, idx, total_loss
